# baseline (speedup 1.0000x reference)
import functools

import jax
import jax.numpy as jnp
from jax import lax
from jax.experimental import pallas as pl
from jax.experimental.pallas import tpu as pltpu
from jax.experimental.pallas import tpu_sc as plsc

N_NODES = 10000
N_EDGES = 320000
N_GRAPHS = 64
IN_CH = 128
HID = 64

NC = 2
NS = 16
NW = NC * NS
CHUNK = 64
NROW = 64
KR = 80
KRG = KR // 2
EROW = KR * CHUNK
E_PAD = NROW * EROW
ROWS0 = 64
J0 = ROWS0 // NS
N_PAD = 10240
RPT = N_PAD // NS
FW = 128
DEGW = 128

_mesh = plsc.VectorSubcoreMesh(
    core_axis_name="c", subcore_axis_name="s", num_cores=NC, num_subcores=NS
)


@functools.partial(
    pl.kernel,
    out_type=jax.ShapeDtypeStruct((NC, N_PAD, DEGW), jnp.float32),
    mesh=_mesh,
    scratch_types=[
        pltpu.VMEM((KR, CHUNK), jnp.int32),
        pltpu.VMEM((CHUNK, DEGW), jnp.float32),
        pltpu.VMEM_SHARED((N_PAD, DEGW), jnp.float32),
    ],
)
def _sc_deg(dst_hbm, ones_hbm, zeros_hbm, out_hbm, dst_v, ones_v, dacc):
    c = lax.axis_index("c")
    s = lax.axis_index("s")
    wid = c * NS + s
    pltpu.sync_copy(ones_hbm, ones_v)
    pltpu.sync_copy(zeros_hbm, dacc.at[pl.ds(s * RPT, RPT)])
    plsc.subcore_barrier()

    def body(k, carry):
        pltpu.sync_copy(ones_v, dacc.at[dst_v.at[k]], add=True)
        return carry

    for j in range(NROW // NW):
        pltpu.sync_copy(dst_hbm.at[wid * (NROW // NW) + j], dst_v)
        lax.fori_loop(0, KR, body, 0)
    plsc.subcore_barrier()
    pltpu.sync_copy(
        dacc.at[pl.ds(s * RPT, RPT)], out_hbm.at[c, pl.ds(s * RPT, RPT)]
    )


@functools.partial(
    pl.kernel,
    out_type=jax.ShapeDtypeStruct((NC, N_PAD, FW), jnp.float32),
    mesh=_mesh,
    scratch_types=[
        pltpu.VMEM((KRG, 2 * CHUNK), jnp.int32),
        pltpu.VMEM((KR, CHUNK), jnp.int32),
        pltpu.VMEM((CHUNK, FW), jnp.float32),
        pltpu.VMEM((CHUNK, FW), jnp.float32),
        pltpu.VMEM_SHARED((N_PAD, FW), jnp.float32),
        pltpu.SemaphoreType.DMA,
        pltpu.SemaphoreType.DMA,
    ],
)
def _sc_agg(
    zs_hbm, src_hbm, dst_hbm, zeros_hbm, out_hbm,
    src_v, dst_v, rows0, rows1, acc, sem0, sem1,
):
    c = lax.axis_index("c")
    s = lax.axis_index("s")
    pltpu.sync_copy(zeros_hbm, acc.at[pl.ds(s * RPT, RPT)])
    plsc.subcore_barrier()

    rows = (rows0, rows1)
    sems = (sem0, sem1)

    def body(i, carry):
        for b in range(2):
            k = i * 2 + b
            pltpu.make_async_copy(
                zs_hbm.at[src_v.at[i, pl.ds(b * CHUNK, CHUNK)]],
                rows[b], sems[b],
            ).wait()
            pltpu.sync_copy(rows[b], acc.at[dst_v.at[k]], add=True)

            @pl.when(k + 2 < KR)
            def _():
                pltpu.async_copy(
                    zs_hbm.at[src_v.at[i + 1, pl.ds(b * CHUNK, CHUNK)]],
                    rows[b], sems[b],
                )

        return carry

    for j in range(J0):
        nrows = jnp.where(c == 0, J0, 0)

        @pl.when(j < nrows)
        def _():
            rid = j * NS + s
            pltpu.sync_copy(src_hbm.at[rid], src_v)
            pltpu.sync_copy(dst_hbm.at[rid], dst_v)
            pltpu.async_copy(
                zs_hbm.at[src_v.at[0, pl.ds(0, CHUNK)]], rows0, sem0
            )
            pltpu.async_copy(
                zs_hbm.at[src_v.at[0, pl.ds(CHUNK, CHUNK)]], rows1, sem1
            )
            lax.fori_loop(0, KR // 2, body, 0)

    plsc.subcore_barrier()
    pltpu.sync_copy(
        acc.at[pl.ds(s * RPT, RPT)], out_hbm.at[c, pl.ds(s * RPT, RPT)]
    )


def _tc1_body(x_ref, w_ref, deg_ref, zs_ref, dis_ref):
    dtot = deg_ref[0, :, 0:1] + deg_ref[1, :, 0:1] + 1.0
    dis = 1.0 / jnp.sqrt(dtot)
    dis64 = jnp.broadcast_to(dis, (N_PAD, HID))
    z = jnp.dot(x_ref[...], w_ref[...], preferred_element_type=jnp.float32)
    zs_ref[...] = jnp.concatenate(
        [z * dis64, jnp.zeros((N_PAD, FW - HID), jnp.float32)], axis=1
    )
    dis_ref[...] = dis64


_tc1 = pl.pallas_call(
    _tc1_body,
    out_shape=[
        jax.ShapeDtypeStruct((N_PAD, FW), jnp.float32),
        jax.ShapeDtypeStruct((N_PAD, HID), jnp.float32),
    ],
)


def _tc_mid_body(agg_ref, zs_ref, dis_ref, w_ref, b_ref, out_ref):
    dis = dis_ref[...]
    agg = agg_ref[0, :, :HID] + agg_ref[1, :, :HID]
    h = dis * agg + dis * zs_ref[:, :HID] + b_ref[...]
    h = jnp.maximum(h, 0.0)
    rows = lax.broadcasted_iota(jnp.int32, (N_PAD, 1), 0)
    h = jnp.where(rows < N_NODES, h, 0.0)
    z = jnp.dot(h, w_ref[...], preferred_element_type=jnp.float32)
    out_ref[...] = jnp.concatenate(
        [z * dis, jnp.zeros((N_PAD, FW - HID), jnp.float32)], axis=1
    )


_tc_mid = pl.pallas_call(
    _tc_mid_body,
    out_shape=jax.ShapeDtypeStruct((N_PAD, FW), jnp.float32),
)


def _tc_fin_body(
    agg_ref, zs_ref, dis_ref, b_ref, batch_ref, wl1_ref, bl1_ref,
    wl2_ref, bl2_ref, out_ref,
):
    dis = dis_ref[...]
    agg = agg_ref[0, :, :HID] + agg_ref[1, :, :HID]
    h = dis * agg + dis * zs_ref[:, :HID] + b_ref[...]
    h = jnp.maximum(h, 0.0)
    gid = lax.broadcasted_iota(jnp.int32, (N_PAD, N_GRAPHS), 1)
    oh = (batch_ref[...] == gid).astype(jnp.float32)
    gsum = lax.dot_general(
        oh, h, (((0,), (0,)), ((), ())), preferred_element_type=jnp.float32
    )
    cnt = jnp.sum(oh, axis=0)[:, None]
    g = gsum / jnp.maximum(cnt, 1.0)
    a = jnp.dot(g, wl1_ref[...], preferred_element_type=jnp.float32)
    a = jnp.maximum(a + bl1_ref[...], 0.0)
    o = jnp.dot(a, wl2_ref[...], preferred_element_type=jnp.float32)
    out_ref[...] = jax.nn.sigmoid(o + bl2_ref[...])


_tc_fin = pl.pallas_call(
    _tc_fin_body,
    out_shape=jax.ShapeDtypeStruct((N_GRAPHS, 1), jnp.float32),
)


def kernel(x, edge_index, batch, W1, b1, W2, b2, W3, b3, Wl1, bl1, Wl2, bl2):
    f32 = jnp.float32
    x_pad = jnp.zeros((N_PAD, IN_CH), f32).at[:N_NODES].set(x)
    src = edge_index[0].astype(jnp.int32)
    dst = edge_index[1].astype(jnp.int32)
    pad = jnp.full((E_PAD - N_EDGES,), N_NODES, jnp.int32)
    src_arr = jnp.concatenate([src, pad]).reshape(NROW, KRG, 2 * CHUNK)
    dst_arr = jnp.concatenate([dst, pad]).reshape(NROW, KR, CHUNK)
    batch_pad = jnp.full((N_PAD, 1), N_GRAPHS, jnp.int32)
    batch_pad = batch_pad.at[:N_NODES, 0].set(batch.astype(jnp.int32))

    zeros_deg = jnp.zeros((RPT, DEGW), f32)
    ones_deg = jnp.ones((CHUNK, DEGW), f32)
    zeros_agg = jnp.zeros((RPT, FW), f32)

    b1r = b1.reshape(1, HID)
    b2r = b2.reshape(1, HID)
    b3r = b3.reshape(1, HID)
    bl1r = bl1.reshape(1, HID // 2)
    bl2r = bl2.reshape(1, 1)

    deg = _sc_deg(dst_arr, ones_deg, zeros_deg)
    zs1, dis64 = _tc1(x_pad, W1, deg)
    agg1 = _sc_agg(zs1, src_arr, dst_arr, zeros_agg)
    zs2 = _tc_mid(agg1, zs1, dis64, W2, b1r)
    agg2 = _sc_agg(zs2, src_arr, dst_arr, zeros_agg)
    zs3 = _tc_mid(agg2, zs2, dis64, W3, b2r)
    agg3 = _sc_agg(zs3, src_arr, dst_arr, zeros_agg)
    out = _tc_fin(agg3, zs3, dis64, b3r, batch_pad, Wl1, bl1r, Wl2, bl2r)
    return out


# speedup vs baseline: 1.1346x; 1.1346x over previous
import functools

import jax
import jax.numpy as jnp
from jax import lax
from jax.experimental import pallas as pl
from jax.experimental.pallas import tpu as pltpu
from jax.experimental.pallas import tpu_sc as plsc

N_NODES = 10000
N_EDGES = 320000
N_GRAPHS = 64
IN_CH = 128
HID = 64

NC = 2
NS = 16
NW = NC * NS
CHUNK = 64
NROW = 128
KR = 40
KRG = KR // 2
EROW = KR * CHUNK
E_PAD = NROW * EROW
ROWS0 = 108
J0 = -(-ROWS0 // NS)
J1 = -(-(NROW - ROWS0) // NS)
N_PAD = 10240
RPT = N_PAD // NS
FW = 128
DEGW = 128

_mesh = plsc.VectorSubcoreMesh(
    core_axis_name="c", subcore_axis_name="s", num_cores=NC, num_subcores=NS
)


@functools.partial(
    pl.kernel,
    out_type=jax.ShapeDtypeStruct((NC, N_PAD, DEGW), jnp.float32),
    mesh=_mesh,
    scratch_types=[
        pltpu.VMEM((KR, CHUNK), jnp.int32),
        pltpu.VMEM((CHUNK, DEGW), jnp.float32),
        pltpu.VMEM_SHARED((N_PAD, DEGW), jnp.float32),
    ],
)
def _sc_deg(dst_hbm, ones_hbm, zeros_hbm, out_hbm, dst_v, ones_v, dacc):
    c = lax.axis_index("c")
    s = lax.axis_index("s")
    wid = c * NS + s
    pltpu.sync_copy(ones_hbm, ones_v)
    pltpu.sync_copy(zeros_hbm, dacc.at[pl.ds(s * RPT, RPT)])
    plsc.subcore_barrier()

    def body(k, carry):
        pltpu.sync_copy(ones_v, dacc.at[dst_v.at[k]], add=True)
        return carry

    for j in range(NROW // NW):
        pltpu.sync_copy(dst_hbm.at[wid * (NROW // NW) + j], dst_v)
        lax.fori_loop(0, KR, body, 0)
    plsc.subcore_barrier()
    pltpu.sync_copy(
        dacc.at[pl.ds(s * RPT, RPT)], out_hbm.at[c, pl.ds(s * RPT, RPT)]
    )


@functools.partial(
    pl.kernel,
    out_type=jax.ShapeDtypeStruct((NC, N_PAD, FW), jnp.float32),
    mesh=_mesh,
    scratch_types=[
        pltpu.VMEM((KRG, 2 * CHUNK), jnp.int32),
        pltpu.VMEM((KR, CHUNK), jnp.int32),
        pltpu.VMEM((CHUNK, FW), jnp.float32),
        pltpu.VMEM((CHUNK, FW), jnp.float32),
        pltpu.VMEM_SHARED((N_PAD, FW), jnp.float32),
        pltpu.SemaphoreType.DMA,
        pltpu.SemaphoreType.DMA,
    ],
)
def _sc_agg(
    zs_hbm, src_hbm, dst_hbm, zeros_hbm, out_hbm,
    src_v, dst_v, rows0, rows1, acc, sem0, sem1,
):
    c = lax.axis_index("c")
    s = lax.axis_index("s")
    pltpu.sync_copy(zeros_hbm, acc.at[pl.ds(s * RPT, RPT)])
    plsc.subcore_barrier()

    rows = (rows0, rows1)
    sems = (sem0, sem1)

    def body(i, carry):
        for b in range(2):
            k = i * 2 + b
            pltpu.make_async_copy(
                zs_hbm.at[src_v.at[i, pl.ds(b * CHUNK, CHUNK)]],
                rows[b], sems[b],
            ).wait()
            pltpu.sync_copy(rows[b], acc.at[dst_v.at[k]], add=True)

            @pl.when(k + 2 < KR)
            def _():
                pltpu.async_copy(
                    zs_hbm.at[src_v.at[i + 1, pl.ds(b * CHUNK, CHUNK)]],
                    rows[b], sems[b],
                )

        return carry

    for j in range(max(J0, J1)):
        rid0 = j * NS + s
        rid1 = ROWS0 + j * NS + s
        rid = jnp.where(c == 0, rid0, rid1)
        valid = jnp.where(c == 0, rid0 < ROWS0, rid1 < NROW)

        @pl.when(valid)
        def _():
            pltpu.sync_copy(src_hbm.at[rid], src_v)
            pltpu.sync_copy(dst_hbm.at[rid], dst_v)
            pltpu.async_copy(
                zs_hbm.at[src_v.at[0, pl.ds(0, CHUNK)]], rows0, sem0
            )
            pltpu.async_copy(
                zs_hbm.at[src_v.at[0, pl.ds(CHUNK, CHUNK)]], rows1, sem1
            )
            lax.fori_loop(0, KR // 2, body, 0)

    plsc.subcore_barrier()
    pltpu.sync_copy(
        acc.at[pl.ds(s * RPT, RPT)], out_hbm.at[c, pl.ds(s * RPT, RPT)]
    )


def _tc1_body(x_ref, w_ref, deg_ref, zs_ref, dis_ref):
    dtot = deg_ref[0, :, 0:1] + deg_ref[1, :, 0:1] + 1.0
    dis = 1.0 / jnp.sqrt(dtot)
    dis64 = jnp.broadcast_to(dis, (N_PAD, HID))
    z = jnp.dot(x_ref[...], w_ref[...], preferred_element_type=jnp.float32)
    zs_ref[...] = jnp.concatenate(
        [z * dis64, jnp.zeros((N_PAD, FW - HID), jnp.float32)], axis=1
    )
    dis_ref[...] = dis64


_tc1 = pl.pallas_call(
    _tc1_body,
    out_shape=[
        jax.ShapeDtypeStruct((N_PAD, FW), jnp.float32),
        jax.ShapeDtypeStruct((N_PAD, HID), jnp.float32),
    ],
)


def _tc_mid_body(agg_ref, zs_ref, dis_ref, w_ref, b_ref, out_ref):
    dis = dis_ref[...]
    agg = agg_ref[0, :, :HID] + agg_ref[1, :, :HID]
    h = dis * agg + dis * zs_ref[:, :HID] + b_ref[...]
    h = jnp.maximum(h, 0.0)
    rows = lax.broadcasted_iota(jnp.int32, (N_PAD, 1), 0)
    h = jnp.where(rows < N_NODES, h, 0.0)
    z = jnp.dot(h, w_ref[...], preferred_element_type=jnp.float32)
    out_ref[...] = jnp.concatenate(
        [z * dis, jnp.zeros((N_PAD, FW - HID), jnp.float32)], axis=1
    )


_tc_mid = pl.pallas_call(
    _tc_mid_body,
    out_shape=jax.ShapeDtypeStruct((N_PAD, FW), jnp.float32),
)


def _tc_fin_body(
    agg_ref, zs_ref, dis_ref, b_ref, batch_ref, wl1_ref, bl1_ref,
    wl2_ref, bl2_ref, out_ref,
):
    dis = dis_ref[...]
    agg = agg_ref[0, :, :HID] + agg_ref[1, :, :HID]
    h = dis * agg + dis * zs_ref[:, :HID] + b_ref[...]
    h = jnp.maximum(h, 0.0)
    gid = lax.broadcasted_iota(jnp.int32, (N_PAD, N_GRAPHS), 1)
    oh = (batch_ref[...] == gid).astype(jnp.float32)
    gsum = lax.dot_general(
        oh, h, (((0,), (0,)), ((), ())), preferred_element_type=jnp.float32
    )
    cnt = jnp.sum(oh, axis=0)[:, None]
    g = gsum / jnp.maximum(cnt, 1.0)
    a = jnp.dot(g, wl1_ref[...], preferred_element_type=jnp.float32)
    a = jnp.maximum(a + bl1_ref[...], 0.0)
    o = jnp.dot(a, wl2_ref[...], preferred_element_type=jnp.float32)
    out_ref[...] = jax.nn.sigmoid(o + bl2_ref[...])


_tc_fin = pl.pallas_call(
    _tc_fin_body,
    out_shape=jax.ShapeDtypeStruct((N_GRAPHS, 1), jnp.float32),
)


def kernel(x, edge_index, batch, W1, b1, W2, b2, W3, b3, Wl1, bl1, Wl2, bl2):
    f32 = jnp.float32
    x_pad = jnp.zeros((N_PAD, IN_CH), f32).at[:N_NODES].set(x)
    src = edge_index[0].astype(jnp.int32)
    dst = edge_index[1].astype(jnp.int32)
    pad = jnp.full((E_PAD - N_EDGES,), N_NODES, jnp.int32)
    src_arr = jnp.concatenate([src, pad]).reshape(NROW, KRG, 2 * CHUNK)
    dst_arr = jnp.concatenate([dst, pad]).reshape(NROW, KR, CHUNK)
    batch_pad = jnp.full((N_PAD, 1), N_GRAPHS, jnp.int32)
    batch_pad = batch_pad.at[:N_NODES, 0].set(batch.astype(jnp.int32))

    zeros_deg = jnp.zeros((RPT, DEGW), f32)
    ones_deg = jnp.ones((CHUNK, DEGW), f32)
    zeros_agg = jnp.zeros((RPT, FW), f32)

    b1r = b1.reshape(1, HID)
    b2r = b2.reshape(1, HID)
    b3r = b3.reshape(1, HID)
    bl1r = bl1.reshape(1, HID // 2)
    bl2r = bl2.reshape(1, 1)

    deg = _sc_deg(dst_arr, ones_deg, zeros_deg)
    zs1, dis64 = _tc1(x_pad, W1, deg)
    agg1 = _sc_agg(zs1, src_arr, dst_arr, zeros_agg)
    zs2 = _tc_mid(agg1, zs1, dis64, W2, b1r)
    agg2 = _sc_agg(zs2, src_arr, dst_arr, zeros_agg)
    zs3 = _tc_mid(agg2, zs2, dis64, W3, b2r)
    agg3 = _sc_agg(zs3, src_arr, dst_arr, zeros_agg)
    out = _tc_fin(agg3, zs3, dis64, b3r, batch_pad, Wl1, bl1r, Wl2, bl2r)
    return out


# speedup vs baseline: 1.2352x; 1.0886x over previous
import functools

import jax
import jax.numpy as jnp
from jax import lax
from jax.experimental import pallas as pl
from jax.experimental.pallas import tpu as pltpu
from jax.experimental.pallas import tpu_sc as plsc

N_NODES = 10000
N_EDGES = 320000
N_GRAPHS = 64
IN_CH = 128
HID = 64

NC = 2
NS = 16
NW = NC * NS
CHUNK = 64
NROW = 128
KR = 40
KRG = KR // 2
EROW = KR * CHUNK
E_PAD = NROW * EROW
ROWS0 = 100
J0 = -(-ROWS0 // NS)
J1 = -(-(NROW - ROWS0) // NS)
N_PAD = 10240
RPT = N_PAD // NS
FW = 128
DEGW = 128

_mesh = plsc.VectorSubcoreMesh(
    core_axis_name="c", subcore_axis_name="s", num_cores=NC, num_subcores=NS
)


@functools.partial(
    pl.kernel,
    out_type=jax.ShapeDtypeStruct((NC, N_PAD, DEGW), jnp.float32),
    mesh=_mesh,
    scratch_types=[
        pltpu.VMEM((KR, CHUNK), jnp.int32),
        pltpu.VMEM((CHUNK, DEGW), jnp.float32),
        pltpu.VMEM_SHARED((N_PAD, DEGW), jnp.float32),
    ],
)
def _sc_deg(dst_hbm, ones_hbm, zeros_hbm, out_hbm, dst_v, ones_v, dacc):
    c = lax.axis_index("c")
    s = lax.axis_index("s")
    wid = c * NS + s
    pltpu.sync_copy(ones_hbm, ones_v)
    pltpu.sync_copy(zeros_hbm, dacc.at[pl.ds(s * RPT, RPT)])
    plsc.subcore_barrier()

    def body(k, carry):
        pltpu.sync_copy(ones_v, dacc.at[dst_v.at[k]], add=True)
        return carry

    for j in range(NROW // NW):
        pltpu.sync_copy(dst_hbm.at[wid * (NROW // NW) + j], dst_v)
        lax.fori_loop(0, KR, body, 0)
    plsc.subcore_barrier()
    pltpu.sync_copy(
        dacc.at[pl.ds(s * RPT, RPT)], out_hbm.at[c, pl.ds(s * RPT, RPT)]
    )


@functools.partial(
    pl.kernel,
    out_type=jax.ShapeDtypeStruct((NC, N_PAD, FW), jnp.float32),
    mesh=_mesh,
    scratch_types=[
        pltpu.VMEM((KRG, 2 * CHUNK), jnp.int32),
        pltpu.VMEM((KR, CHUNK), jnp.int32),
        pltpu.VMEM((CHUNK, FW), jnp.float32),
        pltpu.VMEM((CHUNK, FW), jnp.float32),
        pltpu.VMEM((CHUNK, FW), jnp.float32),
        pltpu.VMEM((CHUNK, FW), jnp.float32),
        pltpu.VMEM_SHARED((N_PAD, FW), jnp.float32),
        pltpu.SemaphoreType.DMA,
        pltpu.SemaphoreType.DMA,
        pltpu.SemaphoreType.DMA,
        pltpu.SemaphoreType.DMA,
        pltpu.SemaphoreType.DMA,
        pltpu.SemaphoreType.DMA,
        pltpu.SemaphoreType.DMA,
        pltpu.SemaphoreType.DMA,
    ],
)
def _sc_agg(
    zs_hbm, src_hbm, dst_hbm, zeros_hbm, out_hbm,
    src_v, dst_v, rows0, rows1, rows2, rows3, acc,
    gsem0, gsem1, gsem2, gsem3, ssem0, ssem1, ssem2, ssem3,
):
    c = lax.axis_index("c")
    s = lax.axis_index("s")
    pltpu.sync_copy(zeros_hbm, acc.at[pl.ds(s * RPT, RPT)])
    plsc.subcore_barrier()

    rows = (rows0, rows1, rows2, rows3)
    gsems = (gsem0, gsem1, gsem2, gsem3)
    ssems = (ssem0, ssem1, ssem2, ssem3)
    NB = 4

    def _gdesc(row, b):
        return pltpu.make_async_copy(
            zs_hbm.at[src_v.at[row, pl.ds((b % 2) * CHUNK, CHUNK)]],
            rows[b], gsems[b],
        )

    def body(i, carry):
        for b in range(NB):
            k = NB * i + b
            _gdesc(2 * i + b // 2, b).wait()
            pltpu.async_copy(
                rows[b], acc.at[dst_v.at[k]], ssems[b], add=True
            )

            @pl.when(k + NB < KR)
            def _():
                pltpu.make_async_copy(
                    rows[b], acc.at[dst_v.at[k]], ssems[b]
                ).wait()
                _gdesc(2 * (i + 1) + b // 2, b).start()

        return carry

    for j in range(max(J0, J1)):
        rid0 = j * NS + s
        rid1 = ROWS0 + j * NS + s
        rid = jnp.where(c == 0, rid0, rid1)
        valid = jnp.where(c == 0, rid0 < ROWS0, rid1 < NROW)

        @pl.when(valid)
        def _():
            pltpu.sync_copy(src_hbm.at[rid], src_v)
            pltpu.sync_copy(dst_hbm.at[rid], dst_v)
            for b in range(NB):
                _gdesc(b // 2, b).start()
            lax.fori_loop(0, KR // NB, body, 0)
            for b in range(NB):
                pltpu.make_async_copy(
                    rows[b], acc.at[dst_v.at[KR - NB + b]], ssems[b]
                ).wait()

    plsc.subcore_barrier()
    pltpu.sync_copy(
        acc.at[pl.ds(s * RPT, RPT)], out_hbm.at[c, pl.ds(s * RPT, RPT)]
    )


def _tc1_body(x_ref, w_ref, deg_ref, zs_ref, dis_ref):
    dtot = deg_ref[0, :, 0:1] + deg_ref[1, :, 0:1] + 1.0
    dis = 1.0 / jnp.sqrt(dtot)
    dis64 = jnp.broadcast_to(dis, (N_PAD, HID))
    z = jnp.dot(x_ref[...], w_ref[...], preferred_element_type=jnp.float32)
    zs_ref[...] = jnp.concatenate(
        [z * dis64, jnp.zeros((N_PAD, FW - HID), jnp.float32)], axis=1
    )
    dis_ref[...] = dis64


_tc1 = pl.pallas_call(
    _tc1_body,
    out_shape=[
        jax.ShapeDtypeStruct((N_PAD, FW), jnp.float32),
        jax.ShapeDtypeStruct((N_PAD, HID), jnp.float32),
    ],
)


def _tc_mid_body(agg_ref, zs_ref, dis_ref, w_ref, b_ref, out_ref):
    dis = dis_ref[...]
    agg = agg_ref[0, :, :HID] + agg_ref[1, :, :HID]
    h = dis * agg + dis * zs_ref[:, :HID] + b_ref[...]
    h = jnp.maximum(h, 0.0)
    rows = lax.broadcasted_iota(jnp.int32, (N_PAD, 1), 0)
    h = jnp.where(rows < N_NODES, h, 0.0)
    z = jnp.dot(h, w_ref[...], preferred_element_type=jnp.float32)
    out_ref[...] = jnp.concatenate(
        [z * dis, jnp.zeros((N_PAD, FW - HID), jnp.float32)], axis=1
    )


_tc_mid = pl.pallas_call(
    _tc_mid_body,
    out_shape=jax.ShapeDtypeStruct((N_PAD, FW), jnp.float32),
)


def _tc_fin_body(
    agg_ref, zs_ref, dis_ref, b_ref, batch_ref, wl1_ref, bl1_ref,
    wl2_ref, bl2_ref, out_ref,
):
    dis = dis_ref[...]
    agg = agg_ref[0, :, :HID] + agg_ref[1, :, :HID]
    h = dis * agg + dis * zs_ref[:, :HID] + b_ref[...]
    h = jnp.maximum(h, 0.0)
    gid = lax.broadcasted_iota(jnp.int32, (N_PAD, N_GRAPHS), 1)
    oh = (batch_ref[...] == gid).astype(jnp.float32)
    gsum = lax.dot_general(
        oh, h, (((0,), (0,)), ((), ())), preferred_element_type=jnp.float32
    )
    cnt = jnp.sum(oh, axis=0)[:, None]
    g = gsum / jnp.maximum(cnt, 1.0)
    a = jnp.dot(g, wl1_ref[...], preferred_element_type=jnp.float32)
    a = jnp.maximum(a + bl1_ref[...], 0.0)
    o = jnp.dot(a, wl2_ref[...], preferred_element_type=jnp.float32)
    out_ref[...] = jax.nn.sigmoid(o + bl2_ref[...])


_tc_fin = pl.pallas_call(
    _tc_fin_body,
    out_shape=jax.ShapeDtypeStruct((N_GRAPHS, 1), jnp.float32),
)


def kernel(x, edge_index, batch, W1, b1, W2, b2, W3, b3, Wl1, bl1, Wl2, bl2):
    f32 = jnp.float32
    x_pad = jnp.zeros((N_PAD, IN_CH), f32).at[:N_NODES].set(x)
    src = edge_index[0].astype(jnp.int32)
    dst = edge_index[1].astype(jnp.int32)
    pad = jnp.full((E_PAD - N_EDGES,), N_NODES, jnp.int32)
    src_arr = jnp.concatenate([src, pad]).reshape(NROW, KRG, 2 * CHUNK)
    dst_arr = jnp.concatenate([dst, pad]).reshape(NROW, KR, CHUNK)
    batch_pad = jnp.full((N_PAD, 1), N_GRAPHS, jnp.int32)
    batch_pad = batch_pad.at[:N_NODES, 0].set(batch.astype(jnp.int32))

    zeros_deg = jnp.zeros((RPT, DEGW), f32)
    ones_deg = jnp.ones((CHUNK, DEGW), f32)
    zeros_agg = jnp.zeros((RPT, FW), f32)

    b1r = b1.reshape(1, HID)
    b2r = b2.reshape(1, HID)
    b3r = b3.reshape(1, HID)
    bl1r = bl1.reshape(1, HID // 2)
    bl2r = bl2.reshape(1, 1)

    deg = _sc_deg(dst_arr, ones_deg, zeros_deg)
    zs1, dis64 = _tc1(x_pad, W1, deg)
    agg1 = _sc_agg(zs1, src_arr, dst_arr, zeros_agg)
    zs2 = _tc_mid(agg1, zs1, dis64, W2, b1r)
    agg2 = _sc_agg(zs2, src_arr, dst_arr, zeros_agg)
    zs3 = _tc_mid(agg2, zs2, dis64, W3, b2r)
    agg3 = _sc_agg(zs3, src_arr, dst_arr, zeros_agg)
    out = _tc_fin(agg3, zs3, dis64, b3r, batch_pad, Wl1, bl1r, Wl2, bl2r)
    return out


# speedup vs baseline: 1.3179x; 1.0669x over previous
import functools

import jax
import jax.numpy as jnp
from jax import lax
from jax.experimental import pallas as pl
from jax.experimental.pallas import tpu as pltpu
from jax.experimental.pallas import tpu_sc as plsc

N_NODES = 10000
N_EDGES = 320000
N_GRAPHS = 64
IN_CH = 128
HID = 64

NC = 2
NS = 16
NW = NC * NS
CHUNK = 64
NROW = 128
KR = 40
KRG = KR // 2
EROW = KR * CHUNK
E_PAD = NROW * EROW
ROWS0 = 112
J0 = -(-ROWS0 // NS)
J1 = -(-(NROW - ROWS0) // NS)
N_PAD = 10240
RPT = N_PAD // NS
FW = 128
DEGW = 128

_mesh = plsc.VectorSubcoreMesh(
    core_axis_name="c", subcore_axis_name="s", num_cores=NC, num_subcores=NS
)


@functools.partial(
    pl.kernel,
    out_type=jax.ShapeDtypeStruct((NC, N_PAD, DEGW), jnp.float32),
    mesh=_mesh,
    scratch_types=[
        pltpu.VMEM((KR, CHUNK), jnp.int32),
        pltpu.VMEM((CHUNK, DEGW), jnp.float32),
        pltpu.VMEM_SHARED((N_PAD, DEGW), jnp.float32),
    ],
)
def _sc_deg(dst_hbm, ones_hbm, zeros_hbm, out_hbm, dst_v, ones_v, dacc):
    c = lax.axis_index("c")
    s = lax.axis_index("s")
    wid = c * NS + s
    pltpu.sync_copy(ones_hbm, ones_v)
    pltpu.sync_copy(zeros_hbm, dacc.at[pl.ds(s * RPT, RPT)])
    plsc.subcore_barrier()

    def body(k, carry):
        pltpu.sync_copy(ones_v, dacc.at[dst_v.at[k]], add=True)
        return carry

    for j in range(NROW // NW):
        pltpu.sync_copy(dst_hbm.at[wid * (NROW // NW) + j], dst_v)
        lax.fori_loop(0, KR, body, 0)
    plsc.subcore_barrier()
    pltpu.sync_copy(
        dacc.at[pl.ds(s * RPT, RPT)], out_hbm.at[c, pl.ds(s * RPT, RPT)]
    )


@functools.partial(
    pl.kernel,
    out_type=jax.ShapeDtypeStruct((NC, N_PAD, FW), jnp.float32),
    mesh=_mesh,
    scratch_types=[
        pltpu.VMEM((KRG, 2 * CHUNK), jnp.int32),
        pltpu.VMEM((KR, CHUNK), jnp.int32),
        pltpu.VMEM((CHUNK, FW), jnp.float32),
        pltpu.VMEM((CHUNK, FW), jnp.float32),
        pltpu.VMEM((CHUNK, FW), jnp.float32),
        pltpu.VMEM((CHUNK, FW), jnp.float32),
        pltpu.VMEM_SHARED((N_PAD, FW), jnp.float32),
        pltpu.SemaphoreType.DMA,
        pltpu.SemaphoreType.DMA,
        pltpu.SemaphoreType.DMA,
        pltpu.SemaphoreType.DMA,
        pltpu.SemaphoreType.DMA,
        pltpu.SemaphoreType.DMA,
        pltpu.SemaphoreType.DMA,
        pltpu.SemaphoreType.DMA,
    ],
)
def _sc_agg(
    zs_hbm, src_hbm, dst_hbm, zeros_hbm, out_hbm,
    src_v, dst_v, rows0, rows1, rows2, rows3, acc,
    gsem0, gsem1, gsem2, gsem3, ssem0, ssem1, ssem2, ssem3,
):
    c = lax.axis_index("c")
    s = lax.axis_index("s")
    pltpu.sync_copy(zeros_hbm, acc.at[pl.ds(s * RPT, RPT)])
    plsc.subcore_barrier()

    rows = (rows0, rows1, rows2, rows3)
    gsems = (gsem0, gsem1, gsem2, gsem3)
    ssems = (ssem0, ssem1, ssem2, ssem3)
    NB = 4

    def _gdesc(row, b):
        return pltpu.make_async_copy(
            zs_hbm.at[src_v.at[row, pl.ds((b % 2) * CHUNK, CHUNK)]],
            rows[b], gsems[b],
        )

    def body(i, carry):
        for b in range(NB):
            k = NB * i + b
            _gdesc(2 * i + b // 2, b).wait()
            pltpu.async_copy(
                rows[b], acc.at[dst_v.at[k]], ssems[b], add=True
            )

            @pl.when(k + NB < KR)
            def _():
                pltpu.make_async_copy(
                    rows[b], acc.at[dst_v.at[k]], ssems[b]
                ).wait()
                _gdesc(2 * (i + 1) + b // 2, b).start()

        return carry

    for j in range(max(J0, J1)):
        rid0 = j * NS + s
        rid1 = ROWS0 + j * NS + s
        rid = jnp.where(c == 0, rid0, rid1)
        valid = jnp.where(c == 0, rid0 < ROWS0, rid1 < NROW)

        @pl.when(valid)
        def _():
            pltpu.sync_copy(src_hbm.at[rid], src_v)
            pltpu.sync_copy(dst_hbm.at[rid], dst_v)
            for b in range(NB):
                _gdesc(b // 2, b).start()
            lax.fori_loop(0, KR // NB, body, 0)
            for b in range(NB):
                pltpu.make_async_copy(
                    rows[b], acc.at[dst_v.at[KR - NB + b]], ssems[b]
                ).wait()

    plsc.subcore_barrier()
    pltpu.sync_copy(
        acc.at[pl.ds(s * RPT, RPT)], out_hbm.at[c, pl.ds(s * RPT, RPT)]
    )


def _tc1_body(x_ref, w_ref, deg_ref, zs_ref, dis_ref):
    dtot = deg_ref[0, :, 0:1] + deg_ref[1, :, 0:1] + 1.0
    dis = 1.0 / jnp.sqrt(dtot)
    dis64 = jnp.broadcast_to(dis, (N_PAD, HID))
    z = jnp.dot(x_ref[...], w_ref[...], preferred_element_type=jnp.float32)
    zs_ref[...] = jnp.concatenate(
        [z * dis64, jnp.zeros((N_PAD, FW - HID), jnp.float32)], axis=1
    )
    dis_ref[...] = dis64


_tc1 = pl.pallas_call(
    _tc1_body,
    out_shape=[
        jax.ShapeDtypeStruct((N_PAD, FW), jnp.float32),
        jax.ShapeDtypeStruct((N_PAD, HID), jnp.float32),
    ],
)


def _tc_mid_body(agg_ref, zs_ref, dis_ref, w_ref, b_ref, out_ref):
    dis = dis_ref[...]
    agg = agg_ref[0, :, :HID] + agg_ref[1, :, :HID]
    h = dis * agg + dis * zs_ref[:, :HID] + b_ref[...]
    h = jnp.maximum(h, 0.0)
    rows = lax.broadcasted_iota(jnp.int32, (N_PAD, 1), 0)
    h = jnp.where(rows < N_NODES, h, 0.0)
    z = jnp.dot(h, w_ref[...], preferred_element_type=jnp.float32)
    out_ref[...] = jnp.concatenate(
        [z * dis, jnp.zeros((N_PAD, FW - HID), jnp.float32)], axis=1
    )


_tc_mid = pl.pallas_call(
    _tc_mid_body,
    out_shape=jax.ShapeDtypeStruct((N_PAD, FW), jnp.float32),
)


def _tc_fin_body(
    agg_ref, zs_ref, dis_ref, b_ref, batch_ref, wl1_ref, bl1_ref,
    wl2_ref, bl2_ref, out_ref,
):
    dis = dis_ref[...]
    agg = agg_ref[0, :, :HID] + agg_ref[1, :, :HID]
    h = dis * agg + dis * zs_ref[:, :HID] + b_ref[...]
    h = jnp.maximum(h, 0.0)
    gid = lax.broadcasted_iota(jnp.int32, (N_PAD, N_GRAPHS), 1)
    oh = (batch_ref[...] == gid).astype(jnp.float32)
    gsum = lax.dot_general(
        oh, h, (((0,), (0,)), ((), ())), preferred_element_type=jnp.float32
    )
    cnt = jnp.sum(oh, axis=0)[:, None]
    g = gsum / jnp.maximum(cnt, 1.0)
    a = jnp.dot(g, wl1_ref[...], preferred_element_type=jnp.float32)
    a = jnp.maximum(a + bl1_ref[...], 0.0)
    o = jnp.dot(a, wl2_ref[...], preferred_element_type=jnp.float32)
    out_ref[...] = jax.nn.sigmoid(o + bl2_ref[...])


_tc_fin = pl.pallas_call(
    _tc_fin_body,
    out_shape=jax.ShapeDtypeStruct((N_GRAPHS, 1), jnp.float32),
)


def kernel(x, edge_index, batch, W1, b1, W2, b2, W3, b3, Wl1, bl1, Wl2, bl2):
    f32 = jnp.float32
    x_pad = jnp.zeros((N_PAD, IN_CH), f32).at[:N_NODES].set(x)
    src = edge_index[0].astype(jnp.int32)
    dst = edge_index[1].astype(jnp.int32)
    pad = jnp.full((E_PAD - N_EDGES,), N_NODES, jnp.int32)
    src_arr = jnp.concatenate([src, pad]).reshape(NROW, KRG, 2 * CHUNK)
    dst_arr = jnp.concatenate([dst, pad]).reshape(NROW, KR, CHUNK)
    batch_pad = jnp.full((N_PAD, 1), N_GRAPHS, jnp.int32)
    batch_pad = batch_pad.at[:N_NODES, 0].set(batch.astype(jnp.int32))

    zeros_deg = jnp.zeros((RPT, DEGW), f32)
    ones_deg = jnp.ones((CHUNK, DEGW), f32)
    zeros_agg = jnp.zeros((RPT, FW), f32)

    b1r = b1.reshape(1, HID)
    b2r = b2.reshape(1, HID)
    b3r = b3.reshape(1, HID)
    bl1r = bl1.reshape(1, HID // 2)
    bl2r = bl2.reshape(1, 1)

    deg = _sc_deg(dst_arr, ones_deg, zeros_deg)
    zs1, dis64 = _tc1(x_pad, W1, deg)
    agg1 = _sc_agg(zs1, src_arr, dst_arr, zeros_agg)
    zs2 = _tc_mid(agg1, zs1, dis64, W2, b1r)
    agg2 = _sc_agg(zs2, src_arr, dst_arr, zeros_agg)
    zs3 = _tc_mid(agg2, zs2, dis64, W3, b2r)
    agg3 = _sc_agg(zs3, src_arr, dst_arr, zeros_agg)
    out = _tc_fin(agg3, zs3, dis64, b3r, batch_pad, Wl1, bl1r, Wl2, bl2r)
    return out
